# baseline (speedup 1.0000x reference)
import functools

import jax
import jax.numpy as jnp
from jax import lax
from jax.experimental import pallas as pl
from jax.experimental.pallas import tpu as pltpu
from jax.experimental.pallas import tpu_sc as plsc

N = 10000
H = 128
ACC = 10240
NC = 2
NS = 16
NW = NC * NS
CHUNK = 32
NSLOT = 8
CH0 = 280
CH1 = 352
E_PAD = NS * (CH0 + CH1) * CHUNK
ROWS_PER_TILE = ACC // NS
BLK = 256
GRID = ACC // BLK
DEPW = 160
DCHUNK = 64
E_PADD = NW * DEPW * DCHUNK


def _sc_mesh():
    return plsc.VectorSubcoreMesh(
        core_axis_name="c", subcore_axis_name="s", num_cores=NC, num_subcores=NS
    )


@functools.cache
def _build_sc_aggregate():
    return pl.kernel(
        _sc_aggregate_body,
        out_type=jax.ShapeDtypeStruct((NC, ACC, H), jnp.float32),
        mesh=_sc_mesh(),
        scratch_types=[
            pltpu.VMEM((2, NSLOT, 2, CHUNK), jnp.int32),
            pltpu.VMEM((NSLOT, CHUNK, H), jnp.float32),
            pltpu.VMEM_SHARED((ACC, H), jnp.float32),
            [pltpu.SemaphoreType.DMA] * NSLOT,
            [pltpu.SemaphoreType.DMA] * NSLOT,
            [pltpu.SemaphoreType.DMA] * NSLOT,
        ],
    )


def _sc_aggregate_body(z_hbm, idx0_hbm, idx1_hbm, out_hbm,
                       ibuf, gbuf, acc_sh, gsems, ssems, isems):
    c = lax.axis_index("c")
    s = lax.axis_index("s")
    rows = pl.ds(s * ROWS_PER_TILE, ROWS_PER_TILE)
    pltpu.sync_copy(z_hbm.at[rows], acc_sh.at[rows])

    def run_edges(idx_hbm, ept):
        for b in range(NSLOT):
            pltpu.async_copy(idx_hbm.at[s, b], ibuf.at[0, b], isems[b])
        for b in range(NSLOT):
            pltpu.make_async_copy(
                idx_hbm.at[s, b], ibuf.at[0, b], isems[b]).wait()
            pltpu.async_copy(z_hbm.at[ibuf.at[0, b, 0]], gbuf.at[b], gsems[b])

        def loop_body(i, _):
            p = lax.rem(i, 2)
            pn = 1 - p
            for b in range(NSLOT):
                j = NSLOT * i + b
                pltpu.make_async_copy(
                    z_hbm.at[ibuf.at[p, b, 0]], gbuf.at[b], gsems[b]).wait()

                @pl.when(j + NSLOT < ept)
                def _():
                    pltpu.async_copy(idx_hbm.at[s, j + NSLOT],
                                     ibuf.at[pn, b], isems[b])

                pltpu.async_copy(gbuf.at[b], acc_sh.at[ibuf.at[p, b, 1]],
                                 ssems[b], add=True)

                @pl.when(j + NSLOT < ept)
                def _():
                    pltpu.make_async_copy(
                        gbuf.at[b], acc_sh.at[ibuf.at[p, b, 1]],
                        ssems[b]).wait()
                    pltpu.make_async_copy(idx_hbm.at[s, j + NSLOT],
                                          ibuf.at[pn, b], isems[b]).wait()
                    pltpu.async_copy(z_hbm.at[ibuf.at[pn, b, 0]],
                                     gbuf.at[b], gsems[b])

                @pl.when(j + NSLOT >= ept)
                def _():
                    pltpu.make_async_copy(
                        gbuf.at[b], acc_sh.at[ibuf.at[p, b, 1]],
                        ssems[b]).wait()

            return 0

        lax.fori_loop(0, ept // NSLOT, loop_body, 0)

    plsc.subcore_barrier()

    @pl.when(c == 0)
    def _():
        run_edges(idx0_hbm, CH0)

    @pl.when(c == 1)
    def _():
        run_edges(idx1_hbm, CH1)

    plsc.subcore_barrier()
    pltpu.sync_copy(acc_sh.at[rows], out_hbm.at[c, rows])


@functools.cache
def _build_sc_degree():
    return pl.kernel(
        _sc_degree_body,
        out_type=jax.ShapeDtypeStruct((NC, ACC, H), jnp.float32),
        mesh=_sc_mesh(),
        scratch_types=[
            pltpu.VMEM((DEPW, DCHUNK), jnp.int32),
            pltpu.VMEM((DCHUNK, H), jnp.float32),
            pltpu.VMEM_SHARED((ACC, H), jnp.float32),
        ],
    )


def _sc_degree_body(zeros_hbm, ones_hbm, dst_hbm, out_hbm, dst_v, ones_v, acc_sh):
    c = lax.axis_index("c")
    s = lax.axis_index("s")
    wid = s * NC + c
    pltpu.sync_copy(dst_hbm.at[wid], dst_v)
    pltpu.sync_copy(ones_hbm, ones_v)
    rows = pl.ds(s * ROWS_PER_TILE, ROWS_PER_TILE)
    pltpu.sync_copy(zeros_hbm.at[rows], acc_sh.at[rows])
    plsc.subcore_barrier()

    def loop_body(j, _):
        pltpu.sync_copy(ones_v, acc_sh.at[dst_v.at[j]], add=True)
        return 0

    lax.fori_loop(0, DEPW, loop_body, 0)
    plsc.subcore_barrier()
    pltpu.sync_copy(acc_sh.at[rows], out_hbm.at[c, rows])


def _dinv_block(d0, d1):
    deg = 1.0 + d0 + d1
    return lax.rsqrt(deg)


def _tc_pre_body(x_ref, w_ref, d0_ref, d1_ref, z_ref):
    dinv = _dinv_block(d0_ref[...], d1_ref[...])
    z_ref[...] = jnp.dot(x_ref[...], w_ref[...],
                         preferred_element_type=jnp.float32) * dinv


def _tc_mid_body(p_ref, zp_ref, d0_ref, d1_ref, b_ref, w_ref, row_ref,
                 z_ref):
    m = pl.program_id(0)
    dinv = _dinv_block(d0_ref[...], d1_ref[...])
    agg = p_ref[0] + p_ref[1] - zp_ref[...]
    h = jnp.maximum(agg * dinv + b_ref[...], 0.0)
    z = (jnp.dot(h, w_ref[...], preferred_element_type=jnp.float32)
         + row_ref[...]) * dinv
    row_ids = m * BLK + lax.broadcasted_iota(jnp.int32, (BLK, 1), 0)
    z_ref[...] = jnp.where(row_ids < N, z, 0.0)


def _tc_final_body(p_ref, zp_ref, d0_ref, d1_ref, b_ref, out_ref):
    dinv = _dinv_block(d0_ref[...], d1_ref[...])
    agg = p_ref[0] + p_ref[1] - zp_ref[...]
    out_ref[...] = jnp.maximum(agg * dinv + b_ref[...], 0.0)


_rows_spec = pl.BlockSpec((BLK, H), lambda m: (m, 0))
_p_spec = pl.BlockSpec((NC, BLK, H), lambda m: (0, m, 0))
_deg_spec = pl.BlockSpec((BLK, 1), lambda m: (m, 0))
_full_spec = pl.BlockSpec((H, H), lambda m: (0, 0))
_row1_spec = pl.BlockSpec((1, H), lambda m: (0, 0))
_out_sds = jax.ShapeDtypeStruct((ACC, H), jnp.float32)

_tc_pre = pl.pallas_call(
    _tc_pre_body, grid=(GRID,),
    in_specs=[_rows_spec, _full_spec, _deg_spec, _deg_spec],
    out_specs=_rows_spec, out_shape=_out_sds)

_tc_mid = pl.pallas_call(
    _tc_mid_body, grid=(GRID,),
    in_specs=[_p_spec, _rows_spec, _deg_spec, _deg_spec, _row1_spec,
              _full_spec, _row1_spec],
    out_specs=_rows_spec, out_shape=_out_sds)

_tc_final = pl.pallas_call(
    _tc_final_body, grid=(GRID,),
    in_specs=[_p_spec, _rows_spec, _deg_spec, _deg_spec, _row1_spec],
    out_specs=_rows_spec, out_shape=_out_sds)


def kernel(x, edge_index, W0, b0, W1, b1, W2, b2, W3, b3):
    f32 = jnp.float32
    src = edge_index[0].astype(jnp.int32)
    dst = edge_index[1].astype(jnp.int32)
    npad = E_PAD - src.shape[0]
    padv = jnp.full((npad,), N, jnp.int32)
    src_f = jnp.concatenate([src, padv])
    dst_f = jnp.concatenate([dst, padv])
    n0 = NS * CH0 * CHUNK
    idx0 = jnp.stack([src_f[:n0].reshape(NS, CH0, CHUNK),
                      dst_f[:n0].reshape(NS, CH0, CHUNK)], axis=2)
    idx1 = jnp.stack([src_f[n0:].reshape(NS, CH1, CHUNK),
                      dst_f[n0:].reshape(NS, CH1, CHUNK)], axis=2)

    npad_d = E_PADD - src.shape[0]
    dst_pd = jnp.concatenate(
        [dst, jnp.full((npad_d,), N, jnp.int32)]).reshape(NW, DEPW, DCHUNK)

    x_pad = jnp.zeros((ACC, H), f32).at[:N].set(x.astype(f32))
    zeros_deg = jnp.zeros((ACC, H), f32)
    ones_chunk = jnp.ones((DCHUNK, H), f32)

    d = _build_sc_degree()(zeros_deg, ones_chunk, dst_pd)
    d0 = d[0, :, :1]
    d1 = d[1, :, :1]

    b0r = b0.reshape(1, H).astype(f32)
    b1r = b1.reshape(1, H).astype(f32)
    b3r = b3.reshape(1, H).astype(f32)
    zero_row = jnp.zeros((1, H), f32)
    row3 = (jax.nn.relu(b2.astype(f32)).reshape(1, H) @ W3.astype(f32)) / 3.0
    W3s = W3.astype(f32) / 3.0

    agg = _build_sc_aggregate()
    z0 = _tc_pre(x_pad, W0.astype(f32), d0, d1)
    p = agg(z0, idx0, idx1)
    z1 = _tc_mid(p, z0, d0, d1, b0r, W1.astype(f32), zero_row)
    p = agg(z1, idx0, idx1)
    z3 = _tc_mid(p, z1, d0, d1, b1r, W3s, row3)
    p = agg(z3, idx0, idx1)
    out = _tc_final(p, z3, d0, d1, b3r)
    return out[:N]


# speedup vs baseline: 1.0912x; 1.0912x over previous
import functools

import jax
import jax.numpy as jnp
from jax import lax
from jax.experimental import pallas as pl
from jax.experimental.pallas import tpu as pltpu
from jax.experimental.pallas import tpu_sc as plsc

N = 10000
H = 128
ACC = 10240
NC = 2
NS = 16
NW = NC * NS
CHUNK = 64
NSLOT = 4
CH0 = 140
CH1 = 176
E_PAD = NS * (CH0 + CH1) * CHUNK
ROWS_PER_TILE = ACC // NS
BLK = 256
GRID = ACC // BLK
DEPW = 160
DCHUNK = 64
E_PADD = NW * DEPW * DCHUNK


def _sc_mesh():
    return plsc.VectorSubcoreMesh(
        core_axis_name="c", subcore_axis_name="s", num_cores=NC, num_subcores=NS
    )


@functools.cache
def _build_sc_aggregate():
    return pl.kernel(
        _sc_aggregate_body,
        out_type=jax.ShapeDtypeStruct((NC, ACC, H), jnp.float32),
        mesh=_sc_mesh(),
        scratch_types=[
            pltpu.VMEM((2, NSLOT, 2, CHUNK), jnp.int32),
            pltpu.VMEM((NSLOT, CHUNK, H), jnp.float32),
            pltpu.VMEM_SHARED((ACC, H), jnp.float32),
            [pltpu.SemaphoreType.DMA] * NSLOT,
            [pltpu.SemaphoreType.DMA] * NSLOT,
            [pltpu.SemaphoreType.DMA] * NSLOT,
        ],
    )


def _sc_aggregate_body(z_hbm, idx0_hbm, idx1_hbm, out_hbm,
                       ibuf, gbuf, acc_sh, gsems, ssems, isems):
    c = lax.axis_index("c")
    s = lax.axis_index("s")
    rows = pl.ds(s * ROWS_PER_TILE, ROWS_PER_TILE)
    pltpu.sync_copy(z_hbm.at[rows], acc_sh.at[rows])

    def run_edges(idx_hbm, ept):
        for b in range(NSLOT):
            pltpu.async_copy(idx_hbm.at[s, b], ibuf.at[0, b], isems[b])
        for b in range(NSLOT):
            pltpu.make_async_copy(
                idx_hbm.at[s, b], ibuf.at[0, b], isems[b]).wait()
            pltpu.async_copy(z_hbm.at[ibuf.at[0, b, 0]], gbuf.at[b], gsems[b])

        def loop_body(i, _):
            p = lax.rem(i, 2)
            pn = 1 - p
            for b in range(NSLOT):
                j = NSLOT * i + b
                pltpu.make_async_copy(
                    z_hbm.at[ibuf.at[p, b, 0]], gbuf.at[b], gsems[b]).wait()

                @pl.when(j + NSLOT < ept)
                def _():
                    pltpu.async_copy(idx_hbm.at[s, j + NSLOT],
                                     ibuf.at[pn, b], isems[b])

                pltpu.async_copy(gbuf.at[b], acc_sh.at[ibuf.at[p, b, 1]],
                                 ssems[b], add=True)

                @pl.when(j + NSLOT < ept)
                def _():
                    pltpu.make_async_copy(
                        gbuf.at[b], acc_sh.at[ibuf.at[p, b, 1]],
                        ssems[b]).wait()
                    pltpu.make_async_copy(idx_hbm.at[s, j + NSLOT],
                                          ibuf.at[pn, b], isems[b]).wait()
                    pltpu.async_copy(z_hbm.at[ibuf.at[pn, b, 0]],
                                     gbuf.at[b], gsems[b])

                @pl.when(j + NSLOT >= ept)
                def _():
                    pltpu.make_async_copy(
                        gbuf.at[b], acc_sh.at[ibuf.at[p, b, 1]],
                        ssems[b]).wait()

            return 0

        lax.fori_loop(0, ept // NSLOT, loop_body, 0)

    plsc.subcore_barrier()

    @pl.when(c == 0)
    def _():
        run_edges(idx0_hbm, CH0)

    @pl.when(c == 1)
    def _():
        run_edges(idx1_hbm, CH1)

    plsc.subcore_barrier()
    pltpu.sync_copy(acc_sh.at[rows], out_hbm.at[c, rows])


@functools.cache
def _build_sc_degree():
    return pl.kernel(
        _sc_degree_body,
        out_type=jax.ShapeDtypeStruct((NC, ACC, H), jnp.float32),
        mesh=_sc_mesh(),
        scratch_types=[
            pltpu.VMEM((DEPW, DCHUNK), jnp.int32),
            pltpu.VMEM((DCHUNK, H), jnp.float32),
            pltpu.VMEM_SHARED((ACC, H), jnp.float32),
        ],
    )


def _sc_degree_body(zeros_hbm, ones_hbm, dst_hbm, out_hbm, dst_v, ones_v, acc_sh):
    c = lax.axis_index("c")
    s = lax.axis_index("s")
    wid = s * NC + c
    pltpu.sync_copy(dst_hbm.at[wid], dst_v)
    pltpu.sync_copy(ones_hbm, ones_v)
    rows = pl.ds(s * ROWS_PER_TILE, ROWS_PER_TILE)
    pltpu.sync_copy(zeros_hbm.at[rows], acc_sh.at[rows])
    plsc.subcore_barrier()

    def loop_body(j, _):
        pltpu.sync_copy(ones_v, acc_sh.at[dst_v.at[j]], add=True)
        return 0

    lax.fori_loop(0, DEPW, loop_body, 0)
    plsc.subcore_barrier()
    pltpu.sync_copy(acc_sh.at[rows], out_hbm.at[c, rows])


def _dinv_block(d0, d1):
    deg = 1.0 + d0 + d1
    return lax.rsqrt(deg)


def _tc_pre_body(x_ref, w_ref, d0_ref, d1_ref, z_ref):
    dinv = _dinv_block(d0_ref[...], d1_ref[...])
    z_ref[...] = jnp.dot(x_ref[...], w_ref[...],
                         preferred_element_type=jnp.float32) * dinv


def _tc_mid_body(p_ref, zp_ref, d0_ref, d1_ref, b_ref, w_ref, row_ref,
                 z_ref):
    m = pl.program_id(0)
    dinv = _dinv_block(d0_ref[...], d1_ref[...])
    agg = p_ref[0] + p_ref[1] - zp_ref[...]
    h = jnp.maximum(agg * dinv + b_ref[...], 0.0)
    z = (jnp.dot(h, w_ref[...], preferred_element_type=jnp.float32)
         + row_ref[...]) * dinv
    row_ids = m * BLK + lax.broadcasted_iota(jnp.int32, (BLK, 1), 0)
    z_ref[...] = jnp.where(row_ids < N, z, 0.0)


def _tc_final_body(p_ref, zp_ref, d0_ref, d1_ref, b_ref, out_ref):
    dinv = _dinv_block(d0_ref[...], d1_ref[...])
    agg = p_ref[0] + p_ref[1] - zp_ref[...]
    out_ref[...] = jnp.maximum(agg * dinv + b_ref[...], 0.0)


_rows_spec = pl.BlockSpec((BLK, H), lambda m: (m, 0))
_p_spec = pl.BlockSpec((NC, BLK, H), lambda m: (0, m, 0))
_deg_spec = pl.BlockSpec((BLK, 1), lambda m: (m, 0))
_full_spec = pl.BlockSpec((H, H), lambda m: (0, 0))
_row1_spec = pl.BlockSpec((1, H), lambda m: (0, 0))
_out_sds = jax.ShapeDtypeStruct((ACC, H), jnp.float32)

_tc_pre = pl.pallas_call(
    _tc_pre_body, grid=(GRID,),
    in_specs=[_rows_spec, _full_spec, _deg_spec, _deg_spec],
    out_specs=_rows_spec, out_shape=_out_sds)

_tc_mid = pl.pallas_call(
    _tc_mid_body, grid=(GRID,),
    in_specs=[_p_spec, _rows_spec, _deg_spec, _deg_spec, _row1_spec,
              _full_spec, _row1_spec],
    out_specs=_rows_spec, out_shape=_out_sds)

_tc_final = pl.pallas_call(
    _tc_final_body, grid=(GRID,),
    in_specs=[_p_spec, _rows_spec, _deg_spec, _deg_spec, _row1_spec],
    out_specs=_rows_spec, out_shape=_out_sds)


def kernel(x, edge_index, W0, b0, W1, b1, W2, b2, W3, b3):
    f32 = jnp.float32
    src = edge_index[0].astype(jnp.int32)
    dst = edge_index[1].astype(jnp.int32)
    npad = E_PAD - src.shape[0]
    padv = jnp.full((npad,), N, jnp.int32)
    src_f = jnp.concatenate([src, padv])
    dst_f = jnp.concatenate([dst, padv])
    n0 = NS * CH0 * CHUNK
    idx0 = jnp.stack([src_f[:n0].reshape(NS, CH0, CHUNK),
                      dst_f[:n0].reshape(NS, CH0, CHUNK)], axis=2)
    idx1 = jnp.stack([src_f[n0:].reshape(NS, CH1, CHUNK),
                      dst_f[n0:].reshape(NS, CH1, CHUNK)], axis=2)

    npad_d = E_PADD - src.shape[0]
    dst_pd = jnp.concatenate(
        [dst, jnp.full((npad_d,), N, jnp.int32)]).reshape(NW, DEPW, DCHUNK)

    x_pad = jnp.zeros((ACC, H), f32).at[:N].set(x.astype(f32))
    zeros_deg = jnp.zeros((ACC, H), f32)
    ones_chunk = jnp.ones((DCHUNK, H), f32)

    d = _build_sc_degree()(zeros_deg, ones_chunk, dst_pd)
    d0 = d[0, :, :1]
    d1 = d[1, :, :1]

    b0r = b0.reshape(1, H).astype(f32)
    b1r = b1.reshape(1, H).astype(f32)
    b3r = b3.reshape(1, H).astype(f32)
    zero_row = jnp.zeros((1, H), f32)
    row3 = (jax.nn.relu(b2.astype(f32)).reshape(1, H) @ W3.astype(f32)) / 3.0
    W3s = W3.astype(f32) / 3.0

    agg = _build_sc_aggregate()
    z0 = _tc_pre(x_pad, W0.astype(f32), d0, d1)
    p = agg(z0, idx0, idx1)
    z1 = _tc_mid(p, z0, d0, d1, b0r, W1.astype(f32), zero_row)
    p = agg(z1, idx0, idx1)
    z3 = _tc_mid(p, z1, d0, d1, b1r, W3s, row3)
    p = agg(z3, idx0, idx1)
    out = _tc_final(p, z3, d0, d1, b3r)
    return out[:N]


# speedup vs baseline: 1.1410x; 1.0456x over previous
import functools

import jax
import jax.numpy as jnp
from jax import lax
from jax.experimental import pallas as pl
from jax.experimental.pallas import tpu as pltpu
from jax.experimental.pallas import tpu_sc as plsc

N = 10000
H = 128
ACC = 10240
NC = 2
NS = 16
NW = NC * NS
CHUNK = 64
NSLOT = 4
CH0 = 180
CH1 = 136
E_PAD = NS * (CH0 + CH1) * CHUNK
ROWS_PER_TILE = ACC // NS
BLK = 256
GRID = ACC // BLK
DEPW = 160
DCHUNK = 64
E_PADD = NW * DEPW * DCHUNK


def _sc_mesh():
    return plsc.VectorSubcoreMesh(
        core_axis_name="c", subcore_axis_name="s", num_cores=NC, num_subcores=NS
    )


@functools.cache
def _build_sc_aggregate():
    return pl.kernel(
        _sc_aggregate_body,
        out_type=jax.ShapeDtypeStruct((NC, ACC, H), jnp.float32),
        mesh=_sc_mesh(),
        scratch_types=[
            pltpu.VMEM((2, NSLOT, 2, CHUNK), jnp.int32),
            pltpu.VMEM((NSLOT, CHUNK, H), jnp.float32),
            pltpu.VMEM_SHARED((ACC, H), jnp.float32),
            [pltpu.SemaphoreType.DMA] * NSLOT,
            [pltpu.SemaphoreType.DMA] * NSLOT,
            [pltpu.SemaphoreType.DMA] * NSLOT,
        ],
    )


def _sc_aggregate_body(z_hbm, idx0_hbm, idx1_hbm, out_hbm,
                       ibuf, gbuf, acc_sh, gsems, ssems, isems):
    c = lax.axis_index("c")
    s = lax.axis_index("s")
    rows = pl.ds(s * ROWS_PER_TILE, ROWS_PER_TILE)
    pltpu.sync_copy(z_hbm.at[rows], acc_sh.at[rows])

    def run_edges(idx_hbm, ept):
        for b in range(NSLOT):
            pltpu.async_copy(idx_hbm.at[s, b], ibuf.at[0, b], isems[b])
        for b in range(NSLOT):
            pltpu.make_async_copy(
                idx_hbm.at[s, b], ibuf.at[0, b], isems[b]).wait()
            pltpu.async_copy(z_hbm.at[ibuf.at[0, b, 0]], gbuf.at[b], gsems[b])

        def loop_body(i, _):
            p = lax.rem(i, 2)
            pn = 1 - p
            for b in range(NSLOT):
                j = NSLOT * i + b
                pltpu.make_async_copy(
                    z_hbm.at[ibuf.at[p, b, 0]], gbuf.at[b], gsems[b]).wait()

                @pl.when(j + NSLOT < ept)
                def _():
                    pltpu.async_copy(idx_hbm.at[s, j + NSLOT],
                                     ibuf.at[pn, b], isems[b])

                pltpu.async_copy(gbuf.at[b], acc_sh.at[ibuf.at[p, b, 1]],
                                 ssems[b], add=True)

                @pl.when(j + NSLOT < ept)
                def _():
                    pltpu.make_async_copy(
                        gbuf.at[b], acc_sh.at[ibuf.at[p, b, 1]],
                        ssems[b]).wait()
                    pltpu.make_async_copy(idx_hbm.at[s, j + NSLOT],
                                          ibuf.at[pn, b], isems[b]).wait()
                    pltpu.async_copy(z_hbm.at[ibuf.at[pn, b, 0]],
                                     gbuf.at[b], gsems[b])

                @pl.when(j + NSLOT >= ept)
                def _():
                    pltpu.make_async_copy(
                        gbuf.at[b], acc_sh.at[ibuf.at[p, b, 1]],
                        ssems[b]).wait()

            return 0

        lax.fori_loop(0, ept // NSLOT, loop_body, 0)

    plsc.subcore_barrier()

    @pl.when(c == 0)
    def _():
        run_edges(idx0_hbm, CH0)

    @pl.when(c == 1)
    def _():
        run_edges(idx1_hbm, CH1)

    plsc.subcore_barrier()
    pltpu.sync_copy(acc_sh.at[rows], out_hbm.at[c, rows])


@functools.cache
def _build_sc_degree():
    return pl.kernel(
        _sc_degree_body,
        out_type=jax.ShapeDtypeStruct((NC, ACC, H), jnp.float32),
        mesh=_sc_mesh(),
        scratch_types=[
            pltpu.VMEM((DEPW, DCHUNK), jnp.int32),
            pltpu.VMEM((DCHUNK, H), jnp.float32),
            pltpu.VMEM_SHARED((ACC, H), jnp.float32),
        ],
    )


def _sc_degree_body(zeros_hbm, ones_hbm, dst_hbm, out_hbm, dst_v, ones_v, acc_sh):
    c = lax.axis_index("c")
    s = lax.axis_index("s")
    wid = s * NC + c
    pltpu.sync_copy(dst_hbm.at[wid], dst_v)
    pltpu.sync_copy(ones_hbm, ones_v)
    rows = pl.ds(s * ROWS_PER_TILE, ROWS_PER_TILE)
    pltpu.sync_copy(zeros_hbm.at[rows], acc_sh.at[rows])
    plsc.subcore_barrier()

    def loop_body(j, _):
        pltpu.sync_copy(ones_v, acc_sh.at[dst_v.at[j]], add=True)
        return 0

    lax.fori_loop(0, DEPW, loop_body, 0)
    plsc.subcore_barrier()
    pltpu.sync_copy(acc_sh.at[rows], out_hbm.at[c, rows])


def _dinv_block(d0, d1):
    deg = 1.0 + d0 + d1
    return lax.rsqrt(deg)


def _tc_pre_body(x_ref, w_ref, d0_ref, d1_ref, z_ref):
    dinv = _dinv_block(d0_ref[...], d1_ref[...])
    z_ref[...] = jnp.dot(x_ref[...], w_ref[...],
                         preferred_element_type=jnp.float32) * dinv


def _tc_mid_body(p_ref, zp_ref, d0_ref, d1_ref, b_ref, w_ref, row_ref,
                 z_ref):
    m = pl.program_id(0)
    dinv = _dinv_block(d0_ref[...], d1_ref[...])
    agg = p_ref[0] + p_ref[1] - zp_ref[...]
    h = jnp.maximum(agg * dinv + b_ref[...], 0.0)
    z = (jnp.dot(h, w_ref[...], preferred_element_type=jnp.float32)
         + row_ref[...]) * dinv
    row_ids = m * BLK + lax.broadcasted_iota(jnp.int32, (BLK, 1), 0)
    z_ref[...] = jnp.where(row_ids < N, z, 0.0)


def _tc_final_body(p_ref, zp_ref, d0_ref, d1_ref, b_ref, out_ref):
    dinv = _dinv_block(d0_ref[...], d1_ref[...])
    agg = p_ref[0] + p_ref[1] - zp_ref[...]
    out_ref[...] = jnp.maximum(agg * dinv + b_ref[...], 0.0)


_rows_spec = pl.BlockSpec((BLK, H), lambda m: (m, 0))
_p_spec = pl.BlockSpec((NC, BLK, H), lambda m: (0, m, 0))
_deg_spec = pl.BlockSpec((BLK, 1), lambda m: (m, 0))
_full_spec = pl.BlockSpec((H, H), lambda m: (0, 0))
_row1_spec = pl.BlockSpec((1, H), lambda m: (0, 0))
_out_sds = jax.ShapeDtypeStruct((ACC, H), jnp.float32)

_tc_pre = pl.pallas_call(
    _tc_pre_body, grid=(GRID,),
    in_specs=[_rows_spec, _full_spec, _deg_spec, _deg_spec],
    out_specs=_rows_spec, out_shape=_out_sds)

_tc_mid = pl.pallas_call(
    _tc_mid_body, grid=(GRID,),
    in_specs=[_p_spec, _rows_spec, _deg_spec, _deg_spec, _row1_spec,
              _full_spec, _row1_spec],
    out_specs=_rows_spec, out_shape=_out_sds)

_tc_final = pl.pallas_call(
    _tc_final_body, grid=(GRID,),
    in_specs=[_p_spec, _rows_spec, _deg_spec, _deg_spec, _row1_spec],
    out_specs=_rows_spec, out_shape=_out_sds)


def kernel(x, edge_index, W0, b0, W1, b1, W2, b2, W3, b3):
    f32 = jnp.float32
    src = edge_index[0].astype(jnp.int32)
    dst = edge_index[1].astype(jnp.int32)
    npad = E_PAD - src.shape[0]
    padv = jnp.full((npad,), N, jnp.int32)
    src_f = jnp.concatenate([src, padv])
    dst_f = jnp.concatenate([dst, padv])
    n0 = NS * CH0 * CHUNK
    idx0 = jnp.stack([src_f[:n0].reshape(NS, CH0, CHUNK),
                      dst_f[:n0].reshape(NS, CH0, CHUNK)], axis=2)
    idx1 = jnp.stack([src_f[n0:].reshape(NS, CH1, CHUNK),
                      dst_f[n0:].reshape(NS, CH1, CHUNK)], axis=2)

    npad_d = E_PADD - src.shape[0]
    dst_pd = jnp.concatenate(
        [dst, jnp.full((npad_d,), N, jnp.int32)]).reshape(NW, DEPW, DCHUNK)

    x_pad = jnp.zeros((ACC, H), f32).at[:N].set(x.astype(f32))
    zeros_deg = jnp.zeros((ACC, H), f32)
    ones_chunk = jnp.ones((DCHUNK, H), f32)

    d = _build_sc_degree()(zeros_deg, ones_chunk, dst_pd)
    d0 = d[0, :, :1]
    d1 = d[1, :, :1]

    b0r = b0.reshape(1, H).astype(f32)
    b1r = b1.reshape(1, H).astype(f32)
    b3r = b3.reshape(1, H).astype(f32)
    zero_row = jnp.zeros((1, H), f32)
    row3 = (jax.nn.relu(b2.astype(f32)).reshape(1, H) @ W3.astype(f32)) / 3.0
    W3s = W3.astype(f32) / 3.0

    agg = _build_sc_aggregate()
    z0 = _tc_pre(x_pad, W0.astype(f32), d0, d1)
    p = agg(z0, idx0, idx1)
    z1 = _tc_mid(p, z0, d0, d1, b0r, W1.astype(f32), zero_row)
    p = agg(z1, idx0, idx1)
    z3 = _tc_mid(p, z1, d0, d1, b1r, W3s, row3)
    p = agg(z3, idx0, idx1)
    out = _tc_final(p, z3, d0, d1, b3r)
    return out[:N]


# speedup vs baseline: 1.2887x; 1.1295x over previous
import functools

import jax
import jax.numpy as jnp
from jax import lax
from jax.experimental import pallas as pl
from jax.experimental.pallas import tpu as pltpu
from jax.experimental.pallas import tpu_sc as plsc

N = 10000
H = 128
ACC = 10240
NC = 2
NS = 16
NW = NC * NS
CHUNK = 64
NSLOT = 4
CH0 = 220
CH1 = 96
E_PAD = NS * (CH0 + CH1) * CHUNK
ROWS_PER_TILE = ACC // NS
BLK = 256
GRID = ACC // BLK
DEPW = 160
DCHUNK = 64
E_PADD = NW * DEPW * DCHUNK


def _sc_mesh():
    return plsc.VectorSubcoreMesh(
        core_axis_name="c", subcore_axis_name="s", num_cores=NC, num_subcores=NS
    )


@functools.cache
def _build_sc_aggregate():
    return pl.kernel(
        _sc_aggregate_body,
        out_type=jax.ShapeDtypeStruct((NC, ACC, H), jnp.float32),
        mesh=_sc_mesh(),
        scratch_types=[
            pltpu.VMEM((2, NSLOT, 2, CHUNK), jnp.int32),
            pltpu.VMEM((NSLOT, CHUNK, H), jnp.float32),
            pltpu.VMEM_SHARED((ACC, H), jnp.float32),
            [pltpu.SemaphoreType.DMA] * NSLOT,
            [pltpu.SemaphoreType.DMA] * NSLOT,
            [pltpu.SemaphoreType.DMA] * NSLOT,
        ],
    )


def _sc_aggregate_body(z_hbm, idx0_hbm, idx1_hbm, out_hbm,
                       ibuf, gbuf, acc_sh, gsems, ssems, isems):
    c = lax.axis_index("c")
    s = lax.axis_index("s")
    rows = pl.ds(s * ROWS_PER_TILE, ROWS_PER_TILE)
    pltpu.sync_copy(z_hbm.at[rows], acc_sh.at[rows])

    def run_edges(idx_hbm, ept):
        for b in range(NSLOT):
            pltpu.async_copy(idx_hbm.at[s, b], ibuf.at[0, b], isems[b])
        for b in range(NSLOT):
            pltpu.make_async_copy(
                idx_hbm.at[s, b], ibuf.at[0, b], isems[b]).wait()
            pltpu.async_copy(z_hbm.at[ibuf.at[0, b, 0]], gbuf.at[b], gsems[b])

        def loop_body(i, _):
            p = lax.rem(i, 2)
            pn = 1 - p
            for b in range(NSLOT):
                j = NSLOT * i + b
                pltpu.make_async_copy(
                    z_hbm.at[ibuf.at[p, b, 0]], gbuf.at[b], gsems[b]).wait()

                @pl.when(j + NSLOT < ept)
                def _():
                    pltpu.async_copy(idx_hbm.at[s, j + NSLOT],
                                     ibuf.at[pn, b], isems[b])

                pltpu.async_copy(gbuf.at[b], acc_sh.at[ibuf.at[p, b, 1]],
                                 ssems[b], add=True)

                @pl.when(j + NSLOT < ept)
                def _():
                    pltpu.make_async_copy(
                        gbuf.at[b], acc_sh.at[ibuf.at[p, b, 1]],
                        ssems[b]).wait()
                    pltpu.make_async_copy(idx_hbm.at[s, j + NSLOT],
                                          ibuf.at[pn, b], isems[b]).wait()
                    pltpu.async_copy(z_hbm.at[ibuf.at[pn, b, 0]],
                                     gbuf.at[b], gsems[b])

                @pl.when(j + NSLOT >= ept)
                def _():
                    pltpu.make_async_copy(
                        gbuf.at[b], acc_sh.at[ibuf.at[p, b, 1]],
                        ssems[b]).wait()

            return 0

        lax.fori_loop(0, ept // NSLOT, loop_body, 0)

    plsc.subcore_barrier()

    @pl.when(c == 0)
    def _():
        run_edges(idx0_hbm, CH0)

    @pl.when(c == 1)
    def _():
        run_edges(idx1_hbm, CH1)

    plsc.subcore_barrier()
    pltpu.sync_copy(acc_sh.at[rows], out_hbm.at[c, rows])


@functools.cache
def _build_sc_degree():
    return pl.kernel(
        _sc_degree_body,
        out_type=jax.ShapeDtypeStruct((NC, ACC, H), jnp.float32),
        mesh=_sc_mesh(),
        scratch_types=[
            pltpu.VMEM((DEPW, DCHUNK), jnp.int32),
            pltpu.VMEM((DCHUNK, H), jnp.float32),
            pltpu.VMEM_SHARED((ACC, H), jnp.float32),
        ],
    )


def _sc_degree_body(zeros_hbm, ones_hbm, dst_hbm, out_hbm, dst_v, ones_v, acc_sh):
    c = lax.axis_index("c")
    s = lax.axis_index("s")
    wid = s * NC + c
    pltpu.sync_copy(dst_hbm.at[wid], dst_v)
    pltpu.sync_copy(ones_hbm, ones_v)
    rows = pl.ds(s * ROWS_PER_TILE, ROWS_PER_TILE)
    pltpu.sync_copy(zeros_hbm.at[rows], acc_sh.at[rows])
    plsc.subcore_barrier()

    def loop_body(j, _):
        pltpu.sync_copy(ones_v, acc_sh.at[dst_v.at[j]], add=True)
        return 0

    lax.fori_loop(0, DEPW, loop_body, 0)
    plsc.subcore_barrier()
    pltpu.sync_copy(acc_sh.at[rows], out_hbm.at[c, rows])


def _dinv_block(d0, d1):
    deg = 1.0 + d0 + d1
    return lax.rsqrt(deg)


def _tc_pre_body(x_ref, w_ref, d0_ref, d1_ref, z_ref):
    dinv = _dinv_block(d0_ref[...], d1_ref[...])
    z_ref[...] = jnp.dot(x_ref[...], w_ref[...],
                         preferred_element_type=jnp.float32) * dinv


def _tc_mid_body(p_ref, zp_ref, d0_ref, d1_ref, b_ref, w_ref, row_ref,
                 z_ref):
    m = pl.program_id(0)
    dinv = _dinv_block(d0_ref[...], d1_ref[...])
    agg = p_ref[0] + p_ref[1] - zp_ref[...]
    h = jnp.maximum(agg * dinv + b_ref[...], 0.0)
    z = (jnp.dot(h, w_ref[...], preferred_element_type=jnp.float32)
         + row_ref[...]) * dinv
    row_ids = m * BLK + lax.broadcasted_iota(jnp.int32, (BLK, 1), 0)
    z_ref[...] = jnp.where(row_ids < N, z, 0.0)


def _tc_final_body(p_ref, zp_ref, d0_ref, d1_ref, b_ref, out_ref):
    dinv = _dinv_block(d0_ref[...], d1_ref[...])
    agg = p_ref[0] + p_ref[1] - zp_ref[...]
    out_ref[...] = jnp.maximum(agg * dinv + b_ref[...], 0.0)


_rows_spec = pl.BlockSpec((BLK, H), lambda m: (m, 0))
_p_spec = pl.BlockSpec((NC, BLK, H), lambda m: (0, m, 0))
_deg_spec = pl.BlockSpec((BLK, 1), lambda m: (m, 0))
_full_spec = pl.BlockSpec((H, H), lambda m: (0, 0))
_row1_spec = pl.BlockSpec((1, H), lambda m: (0, 0))
_out_sds = jax.ShapeDtypeStruct((ACC, H), jnp.float32)

_tc_pre = pl.pallas_call(
    _tc_pre_body, grid=(GRID,),
    in_specs=[_rows_spec, _full_spec, _deg_spec, _deg_spec],
    out_specs=_rows_spec, out_shape=_out_sds)

_tc_mid = pl.pallas_call(
    _tc_mid_body, grid=(GRID,),
    in_specs=[_p_spec, _rows_spec, _deg_spec, _deg_spec, _row1_spec,
              _full_spec, _row1_spec],
    out_specs=_rows_spec, out_shape=_out_sds)

_tc_final = pl.pallas_call(
    _tc_final_body, grid=(GRID,),
    in_specs=[_p_spec, _rows_spec, _deg_spec, _deg_spec, _row1_spec],
    out_specs=_rows_spec, out_shape=_out_sds)


def kernel(x, edge_index, W0, b0, W1, b1, W2, b2, W3, b3):
    f32 = jnp.float32
    src = edge_index[0].astype(jnp.int32)
    dst = edge_index[1].astype(jnp.int32)
    npad = E_PAD - src.shape[0]
    padv = jnp.full((npad,), N, jnp.int32)
    src_f = jnp.concatenate([src, padv])
    dst_f = jnp.concatenate([dst, padv])
    n0 = NS * CH0 * CHUNK
    idx0 = jnp.stack([src_f[:n0].reshape(NS, CH0, CHUNK),
                      dst_f[:n0].reshape(NS, CH0, CHUNK)], axis=2)
    idx1 = jnp.stack([src_f[n0:].reshape(NS, CH1, CHUNK),
                      dst_f[n0:].reshape(NS, CH1, CHUNK)], axis=2)

    npad_d = E_PADD - src.shape[0]
    dst_pd = jnp.concatenate(
        [dst, jnp.full((npad_d,), N, jnp.int32)]).reshape(NW, DEPW, DCHUNK)

    x_pad = jnp.zeros((ACC, H), f32).at[:N].set(x.astype(f32))
    zeros_deg = jnp.zeros((ACC, H), f32)
    ones_chunk = jnp.ones((DCHUNK, H), f32)

    d = _build_sc_degree()(zeros_deg, ones_chunk, dst_pd)
    d0 = d[0, :, :1]
    d1 = d[1, :, :1]

    b0r = b0.reshape(1, H).astype(f32)
    b1r = b1.reshape(1, H).astype(f32)
    b3r = b3.reshape(1, H).astype(f32)
    zero_row = jnp.zeros((1, H), f32)
    row3 = (jax.nn.relu(b2.astype(f32)).reshape(1, H) @ W3.astype(f32)) / 3.0
    W3s = W3.astype(f32) / 3.0

    agg = _build_sc_aggregate()
    z0 = _tc_pre(x_pad, W0.astype(f32), d0, d1)
    p = agg(z0, idx0, idx1)
    z1 = _tc_mid(p, z0, d0, d1, b0r, W1.astype(f32), zero_row)
    p = agg(z1, idx0, idx1)
    z3 = _tc_mid(p, z1, d0, d1, b1r, W3s, row3)
    p = agg(z3, idx0, idx1)
    out = _tc_final(p, z3, d0, d1, b3r)
    return out[:N]


# speedup vs baseline: 1.2917x; 1.0023x over previous
import functools

import jax
import jax.numpy as jnp
from jax import lax
from jax.experimental import pallas as pl
from jax.experimental.pallas import tpu as pltpu
from jax.experimental.pallas import tpu_sc as plsc

N = 10000
H = 128
ACC = 10240
NC = 2
NS = 16
NW = NC * NS
CHUNK = 64
NSLOT = 4
CH0 = 256
CH1 = 60
E_PAD = NS * (CH0 + CH1) * CHUNK
ROWS_PER_TILE = ACC // NS
BLK = 256
GRID = ACC // BLK
DEPW = 160
DCHUNK = 64
E_PADD = NW * DEPW * DCHUNK


def _sc_mesh():
    return plsc.VectorSubcoreMesh(
        core_axis_name="c", subcore_axis_name="s", num_cores=NC, num_subcores=NS
    )


@functools.cache
def _build_sc_aggregate():
    return pl.kernel(
        _sc_aggregate_body,
        out_type=jax.ShapeDtypeStruct((NC, ACC, H), jnp.float32),
        mesh=_sc_mesh(),
        scratch_types=[
            pltpu.VMEM((2, NSLOT, 2, CHUNK), jnp.int32),
            pltpu.VMEM((NSLOT, CHUNK, H), jnp.float32),
            pltpu.VMEM_SHARED((ACC, H), jnp.float32),
            [pltpu.SemaphoreType.DMA] * NSLOT,
            [pltpu.SemaphoreType.DMA] * NSLOT,
            [pltpu.SemaphoreType.DMA] * NSLOT,
        ],
    )


def _sc_aggregate_body(z_hbm, idx0_hbm, idx1_hbm, out_hbm,
                       ibuf, gbuf, acc_sh, gsems, ssems, isems):
    c = lax.axis_index("c")
    s = lax.axis_index("s")
    rows = pl.ds(s * ROWS_PER_TILE, ROWS_PER_TILE)
    pltpu.sync_copy(z_hbm.at[rows], acc_sh.at[rows])

    def run_edges(idx_hbm, ept):
        for b in range(NSLOT):
            pltpu.async_copy(idx_hbm.at[s, b], ibuf.at[0, b], isems[b])
        for b in range(NSLOT):
            pltpu.make_async_copy(
                idx_hbm.at[s, b], ibuf.at[0, b], isems[b]).wait()
            pltpu.async_copy(z_hbm.at[ibuf.at[0, b, 0]], gbuf.at[b], gsems[b])

        def loop_body(i, _):
            p = lax.rem(i, 2)
            pn = 1 - p
            for b in range(NSLOT):
                j = NSLOT * i + b
                pltpu.make_async_copy(
                    z_hbm.at[ibuf.at[p, b, 0]], gbuf.at[b], gsems[b]).wait()

                @pl.when(j + NSLOT < ept)
                def _():
                    pltpu.async_copy(idx_hbm.at[s, j + NSLOT],
                                     ibuf.at[pn, b], isems[b])

                pltpu.async_copy(gbuf.at[b], acc_sh.at[ibuf.at[p, b, 1]],
                                 ssems[b], add=True)

                @pl.when(j + NSLOT < ept)
                def _():
                    pltpu.make_async_copy(
                        gbuf.at[b], acc_sh.at[ibuf.at[p, b, 1]],
                        ssems[b]).wait()
                    pltpu.make_async_copy(idx_hbm.at[s, j + NSLOT],
                                          ibuf.at[pn, b], isems[b]).wait()
                    pltpu.async_copy(z_hbm.at[ibuf.at[pn, b, 0]],
                                     gbuf.at[b], gsems[b])

                @pl.when(j + NSLOT >= ept)
                def _():
                    pltpu.make_async_copy(
                        gbuf.at[b], acc_sh.at[ibuf.at[p, b, 1]],
                        ssems[b]).wait()

            return 0

        lax.fori_loop(0, ept // NSLOT, loop_body, 0)

    plsc.subcore_barrier()

    @pl.when(c == 0)
    def _():
        run_edges(idx0_hbm, CH0)

    @pl.when(c == 1)
    def _():
        run_edges(idx1_hbm, CH1)

    plsc.subcore_barrier()
    pltpu.sync_copy(acc_sh.at[rows], out_hbm.at[c, rows])


@functools.cache
def _build_sc_degree():
    return pl.kernel(
        _sc_degree_body,
        out_type=jax.ShapeDtypeStruct((NC, ACC, H), jnp.float32),
        mesh=_sc_mesh(),
        scratch_types=[
            pltpu.VMEM((DEPW, DCHUNK), jnp.int32),
            pltpu.VMEM((DCHUNK, H), jnp.float32),
            pltpu.VMEM_SHARED((ACC, H), jnp.float32),
        ],
    )


def _sc_degree_body(zeros_hbm, ones_hbm, dst_hbm, out_hbm, dst_v, ones_v, acc_sh):
    c = lax.axis_index("c")
    s = lax.axis_index("s")
    wid = s * NC + c
    pltpu.sync_copy(dst_hbm.at[wid], dst_v)
    pltpu.sync_copy(ones_hbm, ones_v)
    rows = pl.ds(s * ROWS_PER_TILE, ROWS_PER_TILE)
    pltpu.sync_copy(zeros_hbm.at[rows], acc_sh.at[rows])
    plsc.subcore_barrier()

    def loop_body(j, _):
        pltpu.sync_copy(ones_v, acc_sh.at[dst_v.at[j]], add=True)
        return 0

    lax.fori_loop(0, DEPW, loop_body, 0)
    plsc.subcore_barrier()
    pltpu.sync_copy(acc_sh.at[rows], out_hbm.at[c, rows])


def _dinv_block(d0, d1):
    deg = 1.0 + d0 + d1
    return lax.rsqrt(deg)


def _tc_pre_body(x_ref, w_ref, d0_ref, d1_ref, z_ref):
    dinv = _dinv_block(d0_ref[...], d1_ref[...])
    z_ref[...] = jnp.dot(x_ref[...], w_ref[...],
                         preferred_element_type=jnp.float32) * dinv


def _tc_mid_body(p_ref, zp_ref, d0_ref, d1_ref, b_ref, w_ref, row_ref,
                 z_ref):
    m = pl.program_id(0)
    dinv = _dinv_block(d0_ref[...], d1_ref[...])
    agg = p_ref[0] + p_ref[1] - zp_ref[...]
    h = jnp.maximum(agg * dinv + b_ref[...], 0.0)
    z = (jnp.dot(h, w_ref[...], preferred_element_type=jnp.float32)
         + row_ref[...]) * dinv
    row_ids = m * BLK + lax.broadcasted_iota(jnp.int32, (BLK, 1), 0)
    z_ref[...] = jnp.where(row_ids < N, z, 0.0)


def _tc_final_body(p_ref, zp_ref, d0_ref, d1_ref, b_ref, out_ref):
    dinv = _dinv_block(d0_ref[...], d1_ref[...])
    agg = p_ref[0] + p_ref[1] - zp_ref[...]
    out_ref[...] = jnp.maximum(agg * dinv + b_ref[...], 0.0)


_rows_spec = pl.BlockSpec((BLK, H), lambda m: (m, 0))
_p_spec = pl.BlockSpec((NC, BLK, H), lambda m: (0, m, 0))
_deg_spec = pl.BlockSpec((BLK, 1), lambda m: (m, 0))
_full_spec = pl.BlockSpec((H, H), lambda m: (0, 0))
_row1_spec = pl.BlockSpec((1, H), lambda m: (0, 0))
_out_sds = jax.ShapeDtypeStruct((ACC, H), jnp.float32)

_tc_pre = pl.pallas_call(
    _tc_pre_body, grid=(GRID,),
    in_specs=[_rows_spec, _full_spec, _deg_spec, _deg_spec],
    out_specs=_rows_spec, out_shape=_out_sds)

_tc_mid = pl.pallas_call(
    _tc_mid_body, grid=(GRID,),
    in_specs=[_p_spec, _rows_spec, _deg_spec, _deg_spec, _row1_spec,
              _full_spec, _row1_spec],
    out_specs=_rows_spec, out_shape=_out_sds)

_tc_final = pl.pallas_call(
    _tc_final_body, grid=(GRID,),
    in_specs=[_p_spec, _rows_spec, _deg_spec, _deg_spec, _row1_spec],
    out_specs=_rows_spec, out_shape=_out_sds)


def kernel(x, edge_index, W0, b0, W1, b1, W2, b2, W3, b3):
    f32 = jnp.float32
    src = edge_index[0].astype(jnp.int32)
    dst = edge_index[1].astype(jnp.int32)
    npad = E_PAD - src.shape[0]
    padv = jnp.full((npad,), N, jnp.int32)
    src_f = jnp.concatenate([src, padv])
    dst_f = jnp.concatenate([dst, padv])
    n0 = NS * CH0 * CHUNK
    idx0 = jnp.stack([src_f[:n0].reshape(NS, CH0, CHUNK),
                      dst_f[:n0].reshape(NS, CH0, CHUNK)], axis=2)
    idx1 = jnp.stack([src_f[n0:].reshape(NS, CH1, CHUNK),
                      dst_f[n0:].reshape(NS, CH1, CHUNK)], axis=2)

    npad_d = E_PADD - src.shape[0]
    dst_pd = jnp.concatenate(
        [dst, jnp.full((npad_d,), N, jnp.int32)]).reshape(NW, DEPW, DCHUNK)

    x_pad = jnp.zeros((ACC, H), f32).at[:N].set(x.astype(f32))
    zeros_deg = jnp.zeros((ACC, H), f32)
    ones_chunk = jnp.ones((DCHUNK, H), f32)

    d = _build_sc_degree()(zeros_deg, ones_chunk, dst_pd)
    d0 = d[0, :, :1]
    d1 = d[1, :, :1]

    b0r = b0.reshape(1, H).astype(f32)
    b1r = b1.reshape(1, H).astype(f32)
    b3r = b3.reshape(1, H).astype(f32)
    zero_row = jnp.zeros((1, H), f32)
    row3 = (jax.nn.relu(b2.astype(f32)).reshape(1, H) @ W3.astype(f32)) / 3.0
    W3s = W3.astype(f32) / 3.0

    agg = _build_sc_aggregate()
    z0 = _tc_pre(x_pad, W0.astype(f32), d0, d1)
    p = agg(z0, idx0, idx1)
    z1 = _tc_mid(p, z0, d0, d1, b0r, W1.astype(f32), zero_row)
    p = agg(z1, idx0, idx1)
    z3 = _tc_mid(p, z1, d0, d1, b1r, W3s, row3)
    p = agg(z3, idx0, idx1)
    out = _tc_final(p, z3, d0, d1, b3r)
    return out[:N]
